# Initial kernel scaffold; baseline (speedup 1.0000x reference)
import jax
import jax.numpy as jnp
from jax.experimental import pallas as pl


def kernel(x, edge_index, batch, W1, b1, W2, b2, Wfc, bfc):
    raise NotImplementedError("write your pallas kernel here")


# speedup vs baseline: 26.0380x; 26.0380x over previous
import functools

import jax
import jax.numpy as jnp
from jax import lax
from jax.experimental import pallas as pl
from jax.experimental.pallas import tpu as pltpu
from jax.experimental.pallas import tpu_sc as plsc

N = 50000
E = 800000
G = 64

_CHUNK = 128
_NCHUNK = 6272
_MESH = dict(core_axis_name="c", subcore_axis_name="s")


def _deg_body(srcd, ones128, zeros, out, acc, idxb, onesb, zbuf):
    c = lax.axis_index("c")
    s = lax.axis_index("s")
    pltpu.sync_copy(zeros.at[pl.ds(0, 3120)], zbuf)
    pltpu.sync_copy(zbuf, acc.at[pl.ds(s * 3120, 3120)])

    @pl.when(s == 15)
    def _ztail():
        pltpu.sync_copy(zbuf.at[pl.ds(0, 80)], acc.at[pl.ds(49920, 80)])

    pltpu.sync_copy(ones128.at[0], onesb.at[0])
    plsc.subcore_barrier()
    base = c * 3136 + s * 192

    def body(i, _):
        pltpu.sync_copy(srcd.at[pl.ds(base + i * 8, 8)], idxb)
        for j in range(8):
            pltpu.sync_copy(onesb.at[0], acc.at[idxb.at[j]], add=True)
        return 0

    lax.fori_loop(0, 24, body, 0, unroll=False)

    @pl.when(s < 8)
    def _rem():
        pltpu.sync_copy(srcd.at[pl.ds(c * 3136 + 3072 + s * 8, 8)], idxb)
        for j in range(8):
            pltpu.sync_copy(onesb.at[0], acc.at[idxb.at[j]], add=True)

    plsc.subcore_barrier()
    pltpu.sync_copy(acc.at[pl.ds(s * 3120, 3120)], zbuf)
    pltpu.sync_copy(zbuf, out.at[pl.ds(c * N + s * 3120, 3120)])

    @pl.when(s == 15)
    def _otail():
        pltpu.sync_copy(acc.at[pl.ds(49920, 80)], zbuf.at[pl.ds(0, 80)])
        pltpu.sync_copy(zbuf.at[pl.ds(0, 80)], out.at[pl.ds(c * N + 49920, 80)])


def _make_deg():
    return pl.kernel(
        _deg_body,
        out_type=jax.ShapeDtypeStruct((2 * N,), jnp.float32),
        mesh=plsc.VectorSubcoreMesh(**_MESH),
        compiler_params=pltpu.CompilerParams(use_tc_tiling_on_sc=False),
        scratch_types=[
            pltpu.VMEM_SHARED((N + 8,), jnp.float32),
            pltpu.VMEM((8, _CHUNK), jnp.int32),
            pltpu.VMEM((1, _CHUNK), jnp.float32),
            pltpu.VMEM((3120,), jnp.float32),
        ],
    )


def _layer_body(S, ta, tb, src2, dst2, out, acc, sbuf, dbuf, rows_a, rows_b,
                gsem, ssem):
    c = lax.axis_index("c")
    s = lax.axis_index("s")

    def init_from(table):
        for k in range(5):
            pltpu.sync_copy(table.at[pl.ds(s * 3120 + k * 624, 312)],
                            rows_a.at[pl.ds(0, 312)])
            pltpu.sync_copy(table.at[pl.ds(s * 3120 + k * 624 + 312, 312)],
                            rows_b.at[pl.ds(0, 312)])
            pltpu.sync_copy(rows_a.at[pl.ds(0, 312)],
                            acc.at[pl.ds(s * 3120 + k * 624, 312)])
            pltpu.sync_copy(rows_b.at[pl.ds(0, 312)],
                            acc.at[pl.ds(s * 3120 + k * 624 + 312, 312)])

        @pl.when(s == 15)
        def _itail():
            pltpu.sync_copy(table.at[pl.ds(49920, 80)], rows_a.at[pl.ds(0, 80)])
            pltpu.sync_copy(rows_a.at[pl.ds(0, 80)], acc.at[pl.ds(49920, 80)])

    pl.when(c == 0)(lambda: init_from(ta))
    pl.when(c == 1)(lambda: init_from(tb))
    plsc.subcore_barrier()

    cbase = s * 392
    bufs = (rows_a, rows_b)

    def run(table):
        def body(i, _):
            pltpu.sync_copy(src2.at[pl.ds(cbase + i * 8, 8)], sbuf)
            pltpu.sync_copy(dst2.at[pl.ds(cbase + i * 8, 8)], dbuf)
            prev = [None, None]
            for g in range(8 // S):
                buf = bufs[g % 2]
                if prev[g % 2] is not None:
                    for d in prev[g % 2]:
                        d.wait()
                gd = [
                    pltpu.async_copy(table.at[sbuf.at[g * S + j]],
                                     buf.at[pl.ds(j * _CHUNK, _CHUNK)], gsem)
                    for j in range(S)
                ]
                for d in gd:
                    d.wait()
                prev[g % 2] = [
                    pltpu.async_copy(buf.at[pl.ds(j * _CHUNK, _CHUNK)],
                                     acc.at[dbuf.at[g * S + j]], ssem,
                                     add=True)
                    for j in range(S)
                ]
            for pr in prev:
                for d in pr:
                    d.wait()
            return 0

        lax.fori_loop(0, 49, body, 0, unroll=False)

    pl.when(c == 0)(lambda: run(ta))
    pl.when(c == 1)(lambda: run(tb))
    plsc.subcore_barrier()

    for k in range(5):
        pltpu.sync_copy(acc.at[pl.ds(s * 3120 + k * 624, 312)],
                        rows_a.at[pl.ds(0, 312)])
        pltpu.sync_copy(acc.at[pl.ds(s * 3120 + k * 624 + 312, 312)],
                        rows_b.at[pl.ds(0, 312)])
        pltpu.sync_copy(rows_a.at[pl.ds(0, 312)],
                        out.at[pl.ds(c * N + s * 3120 + k * 624, 312)])
        pltpu.sync_copy(rows_b.at[pl.ds(0, 312)],
                        out.at[pl.ds(c * N + s * 3120 + k * 624 + 312, 312)])

    @pl.when(s == 15)
    def _otail():
        pltpu.sync_copy(acc.at[pl.ds(49920, 80)], rows_a.at[pl.ds(0, 80)])
        pltpu.sync_copy(rows_a.at[pl.ds(0, 80)],
                        out.at[pl.ds(c * N + 49920, 80)])


def _make_layer(W):
    S = 4 if W == 16 else 2
    return pl.kernel(
        functools.partial(_layer_body, S),
        out_type=jax.ShapeDtypeStruct((2 * N, W), jnp.float32),
        mesh=plsc.VectorSubcoreMesh(**_MESH),
        compiler_params=pltpu.CompilerParams(use_tc_tiling_on_sc=False),
        scratch_types=[
            pltpu.VMEM_SHARED((N + 8, W), jnp.float32),
            pltpu.VMEM((8, _CHUNK), jnp.int32),
            pltpu.VMEM((8, _CHUNK), jnp.int32),
            pltpu.VMEM((S * _CHUNK, W), jnp.float32),
            pltpu.VMEM((S * _CHUNK, W), jnp.float32),
            pltpu.SemaphoreType.DMA,
            pltpu.SemaphoreType.DMA,
        ],
    )


_RB = 2000
_NBLK = N // _RB


def _mm1_body(x_r, W1_r, b1_r, h_r):
    h_r[...] = lax.dot_general(x_r[...], W1_r[...], (((1,), (1,)), ((), ())),
                               preferred_element_type=jnp.float32) + b1_r[...]


def _make_mm1():
    return pl.pallas_call(
        _mm1_body,
        grid=(_NBLK,),
        in_specs=[
            pl.BlockSpec((_RB, 128), lambda i: (i, 0)),
            pl.BlockSpec((32, 128), lambda i: (0, 0)),
            pl.BlockSpec((1, 32), lambda i: (0, 0)),
        ],
        out_specs=pl.BlockSpec((_RB, 32), lambda i: (i, 0)),
        out_shape=jax.ShapeDtypeStruct((N, 32), jnp.float32),
    )


def _scale_body(h_r, d0_r, d1_r, ua_r, ub_r, dis_r):
    deg = d0_r[...] + d1_r[...] + 1.0
    dis = lax.rsqrt(deg)
    u = dis * h_r[...]
    ua_r[...] = u[:, :16]
    ub_r[...] = u[:, 16:]
    dis_r[...] = dis


def _make_scale():
    return pl.pallas_call(
        _scale_body,
        grid=(_NBLK,),
        in_specs=[
            pl.BlockSpec((_RB, 32), lambda i: (i, 0)),
            pl.BlockSpec((_RB, 1), lambda i: (i, 0)),
            pl.BlockSpec((_RB, 1), lambda i: (i + _NBLK, 0)),
        ],
        out_specs=[
            pl.BlockSpec((_RB, 16), lambda i: (i, 0)),
            pl.BlockSpec((_RB, 16), lambda i: (i, 0)),
            pl.BlockSpec((_RB, 1), lambda i: (i, 0)),
        ],
        out_shape=[
            jax.ShapeDtypeStruct((N, 16), jnp.float32),
            jax.ShapeDtypeStruct((N, 16), jnp.float32),
            jax.ShapeDtypeStruct((N, 1), jnp.float32),
        ],
    )


def _tcB_body(sa_r, sb_r, dis_r, W2_r, b2_r, ua_r, ub_r):
    dis = dis_r[...]
    y1 = jnp.maximum(dis * jnp.concatenate([sa_r[...], sb_r[...]], axis=1), 0.0)
    h = lax.dot_general(y1, W2_r[...], (((1,), (1,)), ((), ())),
                        preferred_element_type=jnp.float32) + b2_r[...]
    u = dis * h
    ua_r[...] = u[:, :32]
    ub_r[...] = u[:, 32:]


def _make_tcB():
    return pl.pallas_call(
        _tcB_body,
        grid=(_NBLK,),
        in_specs=[
            pl.BlockSpec((_RB, 16), lambda i: (i, 0)),
            pl.BlockSpec((_RB, 16), lambda i: (i + _NBLK, 0)),
            pl.BlockSpec((_RB, 1), lambda i: (i, 0)),
            pl.BlockSpec((64, 32), lambda i: (0, 0)),
            pl.BlockSpec((1, 64), lambda i: (0, 0)),
        ],
        out_specs=[
            pl.BlockSpec((_RB, 32), lambda i: (i, 0)),
            pl.BlockSpec((_RB, 32), lambda i: (i, 0)),
        ],
        out_shape=[
            jax.ShapeDtypeStruct((N, 32), jnp.float32),
            jax.ShapeDtypeStruct((N, 32), jnp.float32),
        ],
    )


def _tcC_body(sa_r, sb_r, dis_r, batch_r, Wfc_r, bfc_r, out_r, sum_s, cnt_s):
    i = pl.program_id(0)
    dis = dis_r[...]
    y2 = jnp.maximum(dis * jnp.concatenate([sa_r[...], sb_r[...]], axis=1), 0.0)
    gid = lax.broadcasted_iota(jnp.int32, (_RB, G), 1)
    oh = (batch_r[...] == gid).astype(jnp.float32)
    part = lax.dot_general(oh, y2, (((0,), (0,)), ((), ())),
                           preferred_element_type=jnp.float32)
    cpart = lax.dot_general(oh, jnp.ones((_RB, 1), jnp.float32),
                            (((0,), (0,)), ((), ())),
                            preferred_element_type=jnp.float32)

    @pl.when(i == 0)
    def _init():
        sum_s[...] = jnp.zeros_like(sum_s)
        cnt_s[...] = jnp.zeros_like(cnt_s)

    sum_s[...] += part
    cnt_s[...] += cpart

    @pl.when(i == _NBLK - 1)
    def _fin():
        pooled = sum_s[...] / jnp.clip(cnt_s[...], 1.0, None)
        logits = lax.dot_general(pooled, Wfc_r[...], (((1,), (1,)), ((), ())),
                                 preferred_element_type=jnp.float32) + bfc_r[...]
        m = jnp.max(logits, axis=1, keepdims=True)
        e = jnp.exp(logits - m)
        out_r[...] = e / jnp.sum(e, axis=1, keepdims=True)


def _make_tcC():
    return pl.pallas_call(
        _tcC_body,
        grid=(_NBLK,),
        in_specs=[
            pl.BlockSpec((_RB, 32), lambda i: (i, 0)),
            pl.BlockSpec((_RB, 32), lambda i: (i + _NBLK, 0)),
            pl.BlockSpec((_RB, 1), lambda i: (i, 0)),
            pl.BlockSpec((_RB, 1), lambda i: (i, 0)),
            pl.BlockSpec((10, 64), lambda i: (0, 0)),
            pl.BlockSpec((1, 10), lambda i: (0, 0)),
        ],
        out_specs=pl.BlockSpec((G, 10), lambda i: (0, 0)),
        out_shape=jax.ShapeDtypeStruct((G, 10), jnp.float32),
        scratch_shapes=[
            pltpu.VMEM((G, 64), jnp.float32),
            pltpu.VMEM((G, 1), jnp.float32),
        ],
    )


def kernel(x, edge_index, batch, W1, b1, W2, b2, Wfc, bfc):
    src = edge_index[0]
    dst = edge_index[1]
    npad = _NCHUNK * _CHUNK - E
    src2 = jnp.concatenate([src, jnp.zeros((npad,), jnp.int32)])
    src2 = src2.reshape(_NCHUNK, _CHUNK)
    srcd = jnp.concatenate([src, jnp.full((npad,), N, jnp.int32)])
    srcd = srcd.reshape(_NCHUNK, _CHUNK)
    dst2 = jnp.concatenate([dst, jnp.full((npad,), N, jnp.int32)])
    dst2 = dst2.reshape(_NCHUNK, _CHUNK)
    ones128 = jnp.ones((1, _CHUNK), jnp.float32)
    zeros = jnp.zeros((N,), jnp.float32)

    h1 = _make_mm1()(x, W1, b1.reshape(1, 32))
    degp = _make_deg()(srcd, ones128, zeros)
    dp = degp.reshape(2 * N, 1)
    u1a, u1b, dis = _make_scale()(h1, dp, dp)
    s1 = _make_layer(16)(u1a, u1b, src2, dst2)
    u2a, u2b = _make_tcB()(s1, s1, dis, W2, b2.reshape(1, 64))
    s2 = _make_layer(32)(u2a, u2b, src2, dst2)
    out = _make_tcC()(s2, s2, dis, batch.reshape(N, 1), Wfc, bfc.reshape(1, 10))
    return out


# speedup vs baseline: 26.5069x; 1.0180x over previous
import functools

import jax
import jax.numpy as jnp
from jax import lax
from jax.experimental import pallas as pl
from jax.experimental.pallas import tpu as pltpu
from jax.experimental.pallas import tpu_sc as plsc

N = 50000
E = 800000
G = 64

_CHUNK = 128
_NCHUNK = 6272
_MESH = dict(core_axis_name="c", subcore_axis_name="s")


def _deg_body(srcd, ones128, zeros, out0, out1, acc, idxb, onesb, zbuf):
    c = lax.axis_index("c")
    s = lax.axis_index("s")
    pltpu.sync_copy(zeros.at[pl.ds(0, 3120)], zbuf)
    pltpu.sync_copy(zbuf, acc.at[pl.ds(s * 3120, 3120)])

    @pl.when(s == 15)
    def _ztail():
        pltpu.sync_copy(zbuf.at[pl.ds(0, 80)], acc.at[pl.ds(49920, 80)])

    pltpu.sync_copy(ones128.at[0], onesb.at[0])
    plsc.subcore_barrier()
    base = c * 3136 + s * 192

    def body(i, _):
        pltpu.sync_copy(srcd.at[pl.ds(base + i * 8, 8)], idxb)
        for j in range(8):
            pltpu.sync_copy(onesb.at[0], acc.at[idxb.at[j]], add=True)
        return 0

    lax.fori_loop(0, 24, body, 0, unroll=False)

    @pl.when(s < 8)
    def _rem():
        pltpu.sync_copy(srcd.at[pl.ds(c * 3136 + 3072 + s * 8, 8)], idxb)
        for j in range(8):
            pltpu.sync_copy(onesb.at[0], acc.at[idxb.at[j]], add=True)

    plsc.subcore_barrier()
    pltpu.sync_copy(acc.at[pl.ds(s * 3120, 3120)], zbuf)

    def copy_out(out):
        pltpu.sync_copy(zbuf, out.at[pl.ds(s * 3120, 3120)])

        @pl.when(s == 15)
        def _otail():
            pltpu.sync_copy(acc.at[pl.ds(49920, 80)], zbuf.at[pl.ds(0, 80)])
            pltpu.sync_copy(zbuf.at[pl.ds(0, 80)], out.at[pl.ds(49920, 80)])

    pl.when(c == 0)(lambda: copy_out(out0))
    pl.when(c == 1)(lambda: copy_out(out1))


def _make_deg():
    return pl.kernel(
        _deg_body,
        out_type=[jax.ShapeDtypeStruct((N,), jnp.float32),
                  jax.ShapeDtypeStruct((N,), jnp.float32)],
        mesh=plsc.VectorSubcoreMesh(**_MESH),
        compiler_params=pltpu.CompilerParams(use_tc_tiling_on_sc=False),
        scratch_types=[
            pltpu.VMEM_SHARED((N + 8,), jnp.float32),
            pltpu.VMEM((8, _CHUNK), jnp.int32),
            pltpu.VMEM((1, _CHUNK), jnp.float32),
            pltpu.VMEM((3120,), jnp.float32),
        ],
    )


def _layer_body(S, ta, tb, src2, dst2, out, acc, sbuf, dbuf, rows_a, rows_b,
                gsem, ssem):
    c = lax.axis_index("c")
    s = lax.axis_index("s")

    def init_from(table):
        for k in range(5):
            pltpu.sync_copy(table.at[pl.ds(s * 3120 + k * 624, 312)],
                            rows_a.at[pl.ds(0, 312)])
            pltpu.sync_copy(table.at[pl.ds(s * 3120 + k * 624 + 312, 312)],
                            rows_b.at[pl.ds(0, 312)])
            pltpu.sync_copy(rows_a.at[pl.ds(0, 312)],
                            acc.at[pl.ds(s * 3120 + k * 624, 312)])
            pltpu.sync_copy(rows_b.at[pl.ds(0, 312)],
                            acc.at[pl.ds(s * 3120 + k * 624 + 312, 312)])

        @pl.when(s == 15)
        def _itail():
            pltpu.sync_copy(table.at[pl.ds(49920, 80)], rows_a.at[pl.ds(0, 80)])
            pltpu.sync_copy(rows_a.at[pl.ds(0, 80)], acc.at[pl.ds(49920, 80)])

    pl.when(c == 0)(lambda: init_from(ta))
    pl.when(c == 1)(lambda: init_from(tb))
    plsc.subcore_barrier()

    cbase = s * 392
    bufs = (rows_a, rows_b)

    def run(table):
        def body(i, _):
            pltpu.sync_copy(src2.at[pl.ds(cbase + i * 8, 8)], sbuf)
            pltpu.sync_copy(dst2.at[pl.ds(cbase + i * 8, 8)], dbuf)
            prev = [None, None]
            off = 0
            for g, sz in enumerate(S):
                buf = bufs[g % 2]
                if prev[g % 2] is not None:
                    for d in prev[g % 2]:
                        d.wait()
                gd = [
                    pltpu.async_copy(table.at[sbuf.at[off + j]],
                                     buf.at[pl.ds(j * _CHUNK, _CHUNK)], gsem)
                    for j in range(sz)
                ]
                for d in gd:
                    d.wait()
                prev[g % 2] = [
                    pltpu.async_copy(buf.at[pl.ds(j * _CHUNK, _CHUNK)],
                                     acc.at[dbuf.at[off + j]], ssem,
                                     add=True)
                    for j in range(sz)
                ]
                off += sz
            for pr in prev:
                for d in pr:
                    d.wait()
            return 0

        lax.fori_loop(0, 49, body, 0, unroll=False)

    pl.when(c == 0)(lambda: run(ta))
    pl.when(c == 1)(lambda: run(tb))
    plsc.subcore_barrier()

    for k in range(5):
        pltpu.sync_copy(acc.at[pl.ds(s * 3120 + k * 624, 312)],
                        rows_a.at[pl.ds(0, 312)])
        pltpu.sync_copy(acc.at[pl.ds(s * 3120 + k * 624 + 312, 312)],
                        rows_b.at[pl.ds(0, 312)])
        pltpu.sync_copy(rows_a.at[pl.ds(0, 312)],
                        out.at[pl.ds(c * N + s * 3120 + k * 624, 312)])
        pltpu.sync_copy(rows_b.at[pl.ds(0, 312)],
                        out.at[pl.ds(c * N + s * 3120 + k * 624 + 312, 312)])

    @pl.when(s == 15)
    def _otail():
        pltpu.sync_copy(acc.at[pl.ds(49920, 80)], rows_a.at[pl.ds(0, 80)])
        pltpu.sync_copy(rows_a.at[pl.ds(0, 80)],
                        out.at[pl.ds(c * N + 49920, 80)])


def _make_layer(W):
    S = (4, 4) if W == 16 else (3, 3, 2)
    return pl.kernel(
        functools.partial(_layer_body, S),
        out_type=jax.ShapeDtypeStruct((2 * N, W), jnp.float32),
        mesh=plsc.VectorSubcoreMesh(**_MESH),
        compiler_params=pltpu.CompilerParams(use_tc_tiling_on_sc=False),
        scratch_types=[
            pltpu.VMEM_SHARED((N + 8, W), jnp.float32),
            pltpu.VMEM((8, _CHUNK), jnp.int32),
            pltpu.VMEM((8, _CHUNK), jnp.int32),
            pltpu.VMEM((max(S) * _CHUNK, W), jnp.float32),
            pltpu.VMEM((max(S) * _CHUNK, W), jnp.float32),
            pltpu.SemaphoreType.DMA,
            pltpu.SemaphoreType.DMA,
        ],
    )


_RB = 2000
_NBLK = N // _RB


def _mm1_body(x_r, W1_r, b1_r, h_r):
    h_r[...] = lax.dot_general(x_r[...], W1_r[...], (((1,), (1,)), ((), ())),
                               preferred_element_type=jnp.float32) + b1_r[...]


def _make_mm1():
    return pl.pallas_call(
        _mm1_body,
        grid=(_NBLK,),
        in_specs=[
            pl.BlockSpec((_RB, 128), lambda i: (i, 0)),
            pl.BlockSpec((32, 128), lambda i: (0, 0)),
            pl.BlockSpec((1, 32), lambda i: (0, 0)),
        ],
        out_specs=pl.BlockSpec((_RB, 32), lambda i: (i, 0)),
        out_shape=jax.ShapeDtypeStruct((N, 32), jnp.float32),
    )


def _scale_body(h_r, d0_r, d1_r, ua_r, ub_r, dis_r):
    deg = d0_r[...] + d1_r[...] + 1.0
    dis = lax.rsqrt(deg)
    u = dis * h_r[...]
    ua_r[...] = u[:, :16]
    ub_r[...] = u[:, 16:]
    dis_r[...] = dis


def _make_scale():
    return pl.pallas_call(
        _scale_body,
        grid=(_NBLK,),
        in_specs=[
            pl.BlockSpec((_RB, 32), lambda i: (i, 0)),
            pl.BlockSpec((_RB, 1), lambda i: (i, 0)),
            pl.BlockSpec((_RB, 1), lambda i: (i, 0)),
        ],
        out_specs=[
            pl.BlockSpec((_RB, 16), lambda i: (i, 0)),
            pl.BlockSpec((_RB, 16), lambda i: (i, 0)),
            pl.BlockSpec((_RB, 1), lambda i: (i, 0)),
        ],
        out_shape=[
            jax.ShapeDtypeStruct((N, 16), jnp.float32),
            jax.ShapeDtypeStruct((N, 16), jnp.float32),
            jax.ShapeDtypeStruct((N, 1), jnp.float32),
        ],
    )


def _tcB_body(sa_r, sb_r, dis_r, W2_r, b2_r, ua_r, ub_r):
    dis = dis_r[...]
    y1 = jnp.maximum(dis * jnp.concatenate([sa_r[...], sb_r[...]], axis=1), 0.0)
    h = lax.dot_general(y1, W2_r[...], (((1,), (1,)), ((), ())),
                        preferred_element_type=jnp.float32) + b2_r[...]
    u = dis * h
    ua_r[...] = u[:, :32]
    ub_r[...] = u[:, 32:]


def _make_tcB():
    return pl.pallas_call(
        _tcB_body,
        grid=(_NBLK,),
        in_specs=[
            pl.BlockSpec((_RB, 16), lambda i: (i, 0)),
            pl.BlockSpec((_RB, 16), lambda i: (i + _NBLK, 0)),
            pl.BlockSpec((_RB, 1), lambda i: (i, 0)),
            pl.BlockSpec((64, 32), lambda i: (0, 0)),
            pl.BlockSpec((1, 64), lambda i: (0, 0)),
        ],
        out_specs=[
            pl.BlockSpec((_RB, 32), lambda i: (i, 0)),
            pl.BlockSpec((_RB, 32), lambda i: (i, 0)),
        ],
        out_shape=[
            jax.ShapeDtypeStruct((N, 32), jnp.float32),
            jax.ShapeDtypeStruct((N, 32), jnp.float32),
        ],
    )


def _tcC_body(sa_r, sb_r, dis_r, batch_r, Wfc_r, bfc_r, out_r, sum_s, cnt_s):
    i = pl.program_id(0)
    dis = dis_r[...]
    y2 = jnp.maximum(dis * jnp.concatenate([sa_r[...], sb_r[...]], axis=1), 0.0)
    gid = lax.broadcasted_iota(jnp.int32, (_RB, G), 1)
    oh = (batch_r[...] == gid).astype(jnp.float32)
    part = lax.dot_general(oh, y2, (((0,), (0,)), ((), ())),
                           preferred_element_type=jnp.float32)
    cpart = lax.dot_general(oh, jnp.ones((_RB, 1), jnp.float32),
                            (((0,), (0,)), ((), ())),
                            preferred_element_type=jnp.float32)

    @pl.when(i == 0)
    def _init():
        sum_s[...] = jnp.zeros_like(sum_s)
        cnt_s[...] = jnp.zeros_like(cnt_s)

    sum_s[...] += part
    cnt_s[...] += cpart

    @pl.when(i == _NBLK - 1)
    def _fin():
        pooled = sum_s[...] / jnp.clip(cnt_s[...], 1.0, None)
        logits = lax.dot_general(pooled, Wfc_r[...], (((1,), (1,)), ((), ())),
                                 preferred_element_type=jnp.float32) + bfc_r[...]
        m = jnp.max(logits, axis=1, keepdims=True)
        e = jnp.exp(logits - m)
        out_r[...] = e / jnp.sum(e, axis=1, keepdims=True)


def _make_tcC():
    return pl.pallas_call(
        _tcC_body,
        grid=(_NBLK,),
        in_specs=[
            pl.BlockSpec((_RB, 32), lambda i: (i, 0)),
            pl.BlockSpec((_RB, 32), lambda i: (i + _NBLK, 0)),
            pl.BlockSpec((_RB, 1), lambda i: (i, 0)),
            pl.BlockSpec((_RB, 1), lambda i: (i, 0)),
            pl.BlockSpec((10, 64), lambda i: (0, 0)),
            pl.BlockSpec((1, 10), lambda i: (0, 0)),
        ],
        out_specs=pl.BlockSpec((G, 10), lambda i: (0, 0)),
        out_shape=jax.ShapeDtypeStruct((G, 10), jnp.float32),
        scratch_shapes=[
            pltpu.VMEM((G, 64), jnp.float32),
            pltpu.VMEM((G, 1), jnp.float32),
        ],
    )


def kernel(x, edge_index, batch, W1, b1, W2, b2, Wfc, bfc):
    src = edge_index[0]
    dst = edge_index[1]
    npad = _NCHUNK * _CHUNK - E
    src2 = jnp.concatenate([src, jnp.zeros((npad,), jnp.int32)])
    src2 = src2.reshape(_NCHUNK, _CHUNK)
    srcd = jnp.concatenate([src, jnp.full((npad,), N, jnp.int32)])
    srcd = srcd.reshape(_NCHUNK, _CHUNK)
    dst2 = jnp.concatenate([dst, jnp.full((npad,), N, jnp.int32)])
    dst2 = dst2.reshape(_NCHUNK, _CHUNK)
    ones128 = jnp.ones((1, _CHUNK), jnp.float32)
    zeros = jnp.zeros((N,), jnp.float32)

    h1 = _make_mm1()(x, W1, b1.reshape(1, 32))
    deg0, deg1 = _make_deg()(srcd, ones128, zeros)
    u1a, u1b, dis = _make_scale()(h1, deg0.reshape(N, 1), deg1.reshape(N, 1))
    s1 = _make_layer(16)(u1a, u1b, src2, dst2)
    u2a, u2b = _make_tcB()(s1, s1, dis, W2, b2.reshape(1, 64))
    s2 = _make_layer(32)(u2a, u2b, src2, dst2)
    out = _make_tcC()(s2, s2, dis, batch.reshape(N, 1), Wfc, bfc.reshape(1, 10))
    return out
